# Initial kernel scaffold; baseline (speedup 1.0000x reference)
import jax
import jax.numpy as jnp
from jax.experimental import pallas as pl


def kernel(seq):
    raise NotImplementedError("write your pallas kernel here")


# speedup vs baseline: 2.1265x; 2.1265x over previous
import functools

import jax
import jax.numpy as jnp
from jax import lax
from jax.experimental import pallas as pl
from jax.experimental.pallas import tpu as pltpu
from jax.experimental.pallas import tpu_sc as plsc

L_TOTAL = 2_000_000
CHUNK = 3200
NCHUNK = L_TOTAL // CHUNK
NWORKER = 32
STEPS = -(-NCHUNK // NWORKER)
LANES = 16


def kernel(seq):
    seq = seq.astype(jnp.int32)

    mesh = plsc.VectorSubcoreMesh(core_axis_name="c", subcore_axis_name="s")

    @functools.partial(
        pl.kernel,
        mesh=mesh,
        out_type=jax.ShapeDtypeStruct((4, L_TOTAL), jnp.float32),
        scratch_types=[
            pltpu.VMEM((CHUNK,), jnp.int32),
            pltpu.VMEM((4, CHUNK), jnp.float32),
        ],
    )
    def run(seq_hbm, out_hbm, seq_v, rows_v):
        wid = lax.axis_index("s") * 2 + lax.axis_index("c")

        def step(t, carry):
            chunk = wid + t * NWORKER

            @pl.when(chunk < NCHUNK)
            def _():
                base = chunk * CHUNK
                pltpu.sync_copy(seq_hbm.at[pl.ds(base, CHUNK)], seq_v)

                def vec(j, c2):
                    off = j * LANES
                    s = seq_v[pl.ds(off, LANES)]
                    is_n = s == 4
                    for c in range(4):
                        v = jnp.where(is_n, jnp.float32(0.25),
                                      jnp.where(s == c, jnp.float32(1.0),
                                                jnp.float32(0.0)))
                        rows_v[c, pl.ds(off, LANES)] = v
                    return c2

                lax.fori_loop(0, CHUNK // LANES, vec, 0)

                pltpu.sync_copy(rows_v,
                                out_hbm.at[:, pl.ds(base, CHUNK)])

            return carry

        lax.fori_loop(0, STEPS, step, 0)

    return run(seq)


# speedup vs baseline: 2.8674x; 1.3484x over previous
import functools

import jax
import jax.numpy as jnp
from jax import lax
from jax.experimental import pallas as pl
from jax.experimental.pallas import tpu as pltpu
from jax.experimental.pallas import tpu_sc as plsc

L_TOTAL = 2_000_000
CHUNK = 3200
NCHUNK = L_TOTAL // CHUNK
NWORKER = 32
STEPS = -(-NCHUNK // NWORKER)
LANES = 16
UNROLL = 4


def kernel(seq):
    seq = seq.astype(jnp.int32)

    mesh = plsc.VectorSubcoreMesh(core_axis_name="c", subcore_axis_name="s")

    @functools.partial(
        pl.kernel,
        mesh=mesh,
        out_type=jax.ShapeDtypeStruct((4, L_TOTAL), jnp.float32),
        scratch_types=[
            pltpu.VMEM((2, CHUNK), jnp.int32),
            pltpu.VMEM((2, 4, CHUNK), jnp.float32),
            pltpu.SemaphoreType.DMA,
            pltpu.SemaphoreType.DMA,
            pltpu.SemaphoreType.DMA,
            pltpu.SemaphoreType.DMA,
        ],
    )
    def run(seq_hbm, out_hbm, seq_v, rows_v, in0, in1, out0, out1):
        wid = lax.axis_index("s") * 2 + lax.axis_index("c")
        insem = (in0, in1)
        outsem = (out0, out1)

        def guarded(t, fn):
            lim = NCHUNK - t * NWORKER
            if lim <= 0:
                return
            if lim >= NWORKER:
                fn()
            else:
                pl.when(wid < lim)(fn)

        def in_copy(t):
            base = (wid + t * NWORKER) * CHUNK
            return pltpu.make_async_copy(
                seq_hbm.at[pl.ds(base, CHUNK)], seq_v.at[t % 2],
                insem[t % 2])

        def out_copy(t):
            base = (wid + t * NWORKER) * CHUNK
            return pltpu.make_async_copy(
                rows_v.at[t % 2], out_hbm.at[:, pl.ds(base, CHUNK)],
                outsem[t % 2])

        def compute(t):
            slot = t % 2

            def vec(j, carry):
                for u in range(UNROLL):
                    off = (j * UNROLL + u) * LANES
                    s = seq_v[slot, pl.ds(off, LANES)]
                    fill = jnp.where(s == 4, jnp.float32(0.25),
                                     jnp.float32(0.0))
                    for c in range(4):
                        rows_v[slot, c, pl.ds(off, LANES)] = jnp.where(
                            s == c, jnp.float32(1.0), fill)
                return carry

            lax.fori_loop(0, CHUNK // (LANES * UNROLL), vec, 0)

        guarded(0, lambda: in_copy(0).start())
        guarded(1, lambda: in_copy(1).start())
        for t in range(STEPS):
            guarded(t, lambda t=t: in_copy(t).wait())
            if t >= 2:
                guarded(t - 2, lambda t=t: out_copy(t - 2).wait())
            compute(t)
            guarded(t, lambda t=t: out_copy(t).start())
            guarded(t + 2, lambda t=t: in_copy(t + 2).start())
        guarded(STEPS - 2, lambda: out_copy(STEPS - 2).wait())
        guarded(STEPS - 1, lambda: out_copy(STEPS - 1).wait())

    return run(seq)


# speedup vs baseline: 3.9243x; 1.3686x over previous
import functools

import jax
import jax.numpy as jnp
from jax import lax
from jax.experimental import pallas as pl
from jax.experimental.pallas import tpu as pltpu
from jax.experimental.pallas import tpu_sc as plsc

L_TOTAL = 2_000_000
CHUNK = 3200
NCHUNK = L_TOTAL // CHUNK
NWORKER = 32
STEPS = -(-NCHUNK // NWORKER)
LANES = 16
UNROLL = 4


def kernel(seq):
    seq = seq.astype(jnp.int32)

    mesh = plsc.VectorSubcoreMesh(core_axis_name="c", subcore_axis_name="s")

    @functools.partial(
        pl.kernel,
        mesh=mesh,
        out_type=jax.ShapeDtypeStruct((4, L_TOTAL), jnp.float32),
        scratch_types=[
            pltpu.VMEM((2, CHUNK), jnp.int32),
            pltpu.VMEM((2, 4, CHUNK), jnp.float32),
            pltpu.SemaphoreType.DMA,
            pltpu.SemaphoreType.DMA,
            pltpu.SemaphoreType.DMA,
            pltpu.SemaphoreType.DMA,
        ],
    )
    def run(seq_hbm, out_hbm, seq_v, rows_v, in0, in1, out0, out1):
        wid = lax.axis_index("s") * 2 + lax.axis_index("c")
        insem = (in0, in1)
        outsem = (out0, out1)

        def guarded(t, fn):
            lim = NCHUNK - t * NWORKER
            if lim <= 0:
                return
            if lim >= NWORKER:
                fn()
            else:
                pl.when(wid < lim)(fn)

        def in_copy(t):
            base = (wid + t * NWORKER) * CHUNK
            return pltpu.make_async_copy(
                seq_hbm.at[pl.ds(base, CHUNK)], seq_v.at[t % 2],
                insem[t % 2])

        def out_copy(t):
            base = (wid + t * NWORKER) * CHUNK
            return pltpu.make_async_copy(
                rows_v.at[t % 2], out_hbm.at[:, pl.ds(base, CHUNK)],
                outsem[t % 2])

        def compute(t):
            slot = t % 2

            @plsc.parallel_loop(0, CHUNK, step=LANES, unroll=UNROLL)
            def _(off):
                s = seq_v[slot, pl.ds(off, LANES)]
                fill = jnp.where(s == 4, jnp.float32(0.25),
                                 jnp.float32(0.0))
                for c in range(4):
                    rows_v[slot, c, pl.ds(off, LANES)] = jnp.where(
                        s == c, jnp.float32(1.0), fill)

        guarded(0, lambda: in_copy(0).start())
        guarded(1, lambda: in_copy(1).start())
        for t in range(STEPS):
            guarded(t, lambda t=t: in_copy(t).wait())
            if t >= 2:
                guarded(t - 2, lambda t=t: out_copy(t - 2).wait())
            compute(t)
            guarded(t, lambda t=t: out_copy(t).start())
            guarded(t + 2, lambda t=t: in_copy(t + 2).start())
        guarded(STEPS - 2, lambda: out_copy(STEPS - 2).wait())
        guarded(STEPS - 1, lambda: out_copy(STEPS - 1).wait())

    return run(seq)


# speedup vs baseline: 4.2497x; 1.0829x over previous
import functools

import jax
import jax.numpy as jnp
from jax import lax
from jax.experimental import pallas as pl
from jax.experimental.pallas import tpu as pltpu
from jax.experimental.pallas import tpu_sc as plsc

L_TOTAL = 2_000_000
CHUNK = 3200
NCHUNK = L_TOTAL // CHUNK
NWORKER = 32
STEPS = -(-NCHUNK // NWORKER)
LANES = 16
UNROLL = 4


def kernel(seq):
    seq = seq.astype(jnp.int32)

    mesh = plsc.VectorSubcoreMesh(core_axis_name="c", subcore_axis_name="s")

    @functools.partial(
        pl.kernel,
        mesh=mesh,
        out_type=jax.ShapeDtypeStruct((4, L_TOTAL), jnp.float32),
        scratch_types=[
            pltpu.VMEM((2, CHUNK), jnp.int32),
            pltpu.VMEM((2, 4, CHUNK), jnp.float32),
            pltpu.SemaphoreType.DMA,
            pltpu.SemaphoreType.DMA,
            pltpu.SemaphoreType.DMA,
            pltpu.SemaphoreType.DMA,
        ],
    )
    def run(seq_hbm, out_hbm, seq_v, rows_v, in0, in1, out0, out1):
        wid = lax.axis_index("s") * 2 + lax.axis_index("c")
        insem = (in0, in1)
        outsem = (out0, out1)

        def in_copy(t, slot):
            base = (wid + t * NWORKER) * CHUNK
            return pltpu.make_async_copy(
                seq_hbm.at[pl.ds(base, CHUNK)], seq_v.at[slot], insem[slot])

        def out_copy(t, slot):
            base = (wid + t * NWORKER) * CHUNK
            return pltpu.make_async_copy(
                rows_v.at[slot], out_hbm.at[:, pl.ds(base, CHUNK)],
                outsem[slot])

        def compute(slot):
            @plsc.parallel_loop(0, CHUNK, step=LANES, unroll=UNROLL)
            def _(off):
                s = seq_v[slot, pl.ds(off, LANES)]
                fill = jnp.where(s == 4, jnp.float32(0.25),
                                 jnp.float32(0.0))
                for c in range(4):
                    rows_v[slot, c, pl.ds(off, LANES)] = jnp.where(
                        s == c, jnp.float32(1.0), fill)

        def valid(t):
            return wid + t * NWORKER < NCHUNK

        in_copy(0, 0).start()
        in_copy(1, 1).start()

        def round_(r, carry):
            for p in range(2):
                t = r * 2 + p
                pl.when(valid(t))(lambda: in_copy(t, p).wait())
                pl.when(r >= 1)(lambda: out_copy(t - 2, p).wait())
                compute(p)
                pl.when(valid(t))(lambda: out_copy(t, p).start())
                pl.when(jnp.logical_and(r < STEPS // 2 - 1, valid(t + 2)))(
                    lambda: in_copy(t + 2, p).start())
            return carry

        lax.fori_loop(0, STEPS // 2, round_, 0)

        out_copy(STEPS - 2, 0).wait()
        pl.when(valid(STEPS - 1))(lambda: out_copy(STEPS - 1, 1).wait())

    return run(seq)
